# baseline (speedup 1.0000x reference)
import functools

import jax
import jax.numpy as jnp
from jax import lax
from jax.experimental import pallas as pl
from jax.experimental.pallas import tpu as pltpu
from jax.experimental.pallas import tpu_sc as plsc


@functools.cache
def _build(Bb, H, V, D):
    info = plsc.get_sparse_core_info()
    NC, NS, L = info.num_cores, info.num_subcores, info.num_lanes
    NW = NC * NS
    BCOL = Bb // NW
    assert BCOL == 128
    W = 4 * D
    CH = 2
    n_ch = H // CH
    CR = CH * BCOL
    mesh = plsc.VectorSubcoreMesh(core_axis_name="c", subcore_axis_name="s")

    @functools.partial(
        pl.kernel,
        out_type=jax.ShapeDtypeStruct((H * D, Bb), jnp.float32),
        mesh=mesh,
        scratch_types=[
            pltpu.VMEM((H, BCOL), jnp.int32),
            pltpu.VMEM((H * BCOL,), jnp.int32),
            pltpu.VMEM((2, CR, W), jnp.float32),
            pltpu.VMEM((2, CH * D, BCOL), jnp.float32),
        ]
        + [pltpu.SemaphoreType.DMA] * 4,
        compiler_params=pltpu.CompilerParams(needs_layout_passes=False),
    )
    def k(idxT_hbm, table_hbm, out_hbm, idxT_v, idxS_v, rows_v, out_v, *sems):
        gsems, osems = sems[:2], sems[2:]
        wid = lax.axis_index("s") * NC + lax.axis_index("c")
        col0 = wid * BCOL
        pltpu.sync_copy(idxT_hbm.at[:, pl.ds(col0, BCOL)], idxT_v)

        @pl.loop(0, H)
        def _shift(r):
            for g in range(BCOL // L):
                idxS_v[pl.ds(r * BCOL + g * L, L)] = (
                    idxT_v[r, pl.ds(g * L, L)] & (VP - 1))

        def fire_gather(c, b):
            return pltpu.async_copy(
                table_hbm.at[idxS_v.at[pl.ds(c * CR, CR)]], rows_v.at[b], gsems[b])

        def fire_out(c, b):
            return pltpu.async_copy(
                out_v.at[b],
                out_hbm.at[pl.ds(c * (CH * D), CH * D), pl.ds(col0, BCOL)],
                osems[b])

        def wait_gather(b):
            pltpu.make_async_copy(
                table_hbm.at[idxS_v.at[pl.ds(0, CR)]], rows_v.at[b], gsems[b]).wait()

        def wait_out(b):
            pltpu.make_async_copy(
                out_v.at[b],
                out_hbm.at[pl.ds(0, CH * D), pl.ds(col0, BCOL)],
                osems[b]).wait()

        def extract(c, b):
            rows2d = rows_v.at[b]
            outb = out_v.at[b]
            for d in range(CH):
                h = c * CH + d
                for g in range(BCOL // L):
                    bvec = lax.iota(jnp.int32, L) + (d * BCOL + g * L)
                    mb = (idxT_v[h, pl.ds(g * L, L)] >> 18) * D
                    for j in range(D):
                        v = plsc.load_gather(rows2d, [bvec, mb + j])
                        outb[d * D + j, pl.ds(g * L, L)] = v

        fire_gather(0, 0)
        fire_gather(1, 1)

        @pl.loop(0, n_ch // 2)
        def _grp(gg):
            for db in range(2):
                c = gg * 2 + db
                wait_gather(db)

                @pl.when(gg > 0)
                def _():
                    wait_out(db)

                extract(c, db)
                fire_out(c, db)

                if db == 0:
                    fire_gather(c + 2, db)
                else:
                    @pl.when(gg < n_ch // 2 - 1)
                    def _():
                        fire_gather(c + 2, db)

        c_last = n_ch - 1
        wait_gather(0)
        wait_out(0)
        extract(c_last, 0)
        fire_out(c_last, 0)
        wait_out(0)
        wait_out(1)

    return k


VP = 1 << 18


@functools.cache
def _tc_relayout(V, D):
    BL = 8192
    n_rb = VP // BL

    def body(t0, t1, t2, t3, o_ref):
        for m, t in enumerate((t0, t1, t2, t3)):
            o_ref[:, m * D:(m + 1) * D] = t[...].T

    def mk_spec(m, V):
        last = (V - 1) // BL

        def imap(rb, m=m, last=last):
            return (0, jnp.minimum(m * n_rb + rb, last))

        return pl.BlockSpec((D, BL), imap)

    return pl.pallas_call(
        body,
        grid=(n_rb,),
        in_specs=[mk_spec(m, V) for m in range(4)],
        out_specs=pl.BlockSpec((BL, 4 * D), lambda rb: (rb, 0)),
        out_shape=jax.ShapeDtypeStruct((VP, 4 * D), jnp.float32),
    )


def kernel(inputs, table):
    Bb, H = inputs.shape
    V, D = table.shape
    tt = table.T
    table4 = _tc_relayout(V, D)(tt, tt, tt, tt)
    out2d = _build(Bb, H, V, D)(inputs.T, table4)
    return out2d.reshape(H, D, Bb).transpose(2, 0, 1)


# speedup vs baseline: 1.2388x; 1.2388x over previous
import functools

import jax
import jax.numpy as jnp
from jax import lax
from jax.experimental import pallas as pl
from jax.experimental.pallas import tpu as pltpu
from jax.experimental.pallas import tpu_sc as plsc


@functools.cache
def _build(Bb, H, V, D):
    info = plsc.get_sparse_core_info()
    NC, NS, L = info.num_cores, info.num_subcores, info.num_lanes
    NW = NC * NS
    BCOL = Bb // NW
    assert BCOL == 128
    W = 4 * D
    CH = 2
    n_ch = H // CH
    CR = CH * BCOL
    mesh = plsc.VectorSubcoreMesh(core_axis_name="c", subcore_axis_name="s")

    @functools.partial(
        pl.kernel,
        out_type=jax.ShapeDtypeStruct((H * D, Bb), jnp.float32),
        mesh=mesh,
        scratch_types=[
            pltpu.VMEM((H, BCOL), jnp.int32),
            pltpu.VMEM((H * BCOL,), jnp.int32),
            pltpu.VMEM((2, CR, W), jnp.float32),
            pltpu.VMEM((2, CH * D, BCOL), jnp.float32),
        ]
        + [pltpu.SemaphoreType.DMA] * 4,
        compiler_params=pltpu.CompilerParams(needs_layout_passes=False),
    )
    def k(idxT_hbm, table_hbm, out_hbm, idxT_v, idxS_v, rows_v, out_v, *sems):
        gsems, osems = sems[:2], sems[2:]
        wid = lax.axis_index("s") * NC + lax.axis_index("c")
        col0 = wid * BCOL
        pltpu.sync_copy(idxT_hbm.at[:, pl.ds(col0, BCOL)], idxT_v)

        @pl.loop(0, H)
        def _shift(r):
            for g in range(BCOL // L):
                idxS_v[pl.ds(r * BCOL + g * L, L)] = (
                    idxT_v[r, pl.ds(g * L, L)] & (VP - 1))

        def fire_gather(c, b):
            return pltpu.async_copy(
                table_hbm.at[idxS_v.at[pl.ds(c * CR, CR)]], rows_v.at[b], gsems[b])

        def fire_out(c, b):
            return pltpu.async_copy(
                out_v.at[b],
                out_hbm.at[pl.ds(c * (CH * D), CH * D), pl.ds(col0, BCOL)],
                osems[b])

        def wait_gather(b):
            pltpu.make_async_copy(
                table_hbm.at[idxS_v.at[pl.ds(0, CR)]], rows_v.at[b], gsems[b]).wait()

        def wait_out(b):
            pltpu.make_async_copy(
                out_v.at[b],
                out_hbm.at[pl.ds(0, CH * D), pl.ds(col0, BCOL)],
                osems[b]).wait()

        def extract(c, b):
            rows2d = rows_v.at[b]
            outb = out_v.at[b]
            lvec = lax.iota(jnp.int32, L)
            rsel = lvec >> 2
            qsel = lvec & 3
            for d in range(CH):
                h = c * CH + d
                hvec = jnp.full((L,), h, jnp.int32)

                @pl.loop(0, BCOL // 4)
                def _b4(b4, d=d, hvec=hvec):
                    bv = (b4 * 4) + rsel
                    rowv = (d * BCOL) + bv
                    mbv = (plsc.load_gather(idxT_v, [hvec, bv]) >> 18) * D
                    for j0 in range(0, D, 4):
                        v = plsc.load_gather(rows2d, [rowv, mbv + (j0 + qsel)])
                        plsc.store_scatter(outb, [(d * D + j0) + qsel, bv], v)

        fire_gather(0, 0)
        fire_gather(1, 1)

        @pl.loop(0, n_ch // 2)
        def _grp(gg):
            for db in range(2):
                c = gg * 2 + db
                wait_gather(db)

                @pl.when(gg > 0)
                def _():
                    wait_out(db)

                extract(c, db)
                fire_out(c, db)

                if db == 0:
                    fire_gather(c + 2, db)
                else:
                    @pl.when(gg < n_ch // 2 - 1)
                    def _():
                        fire_gather(c + 2, db)

        c_last = n_ch - 1
        wait_gather(0)
        wait_out(0)
        extract(c_last, 0)
        fire_out(c_last, 0)
        wait_out(0)
        wait_out(1)

    return k


VP = 1 << 18


@functools.cache
def _tc_relayout(V, D):
    BL = 8192
    n_rb = VP // BL

    def body(t0, t1, t2, t3, o_ref):
        for m, t in enumerate((t0, t1, t2, t3)):
            o_ref[:, m * D:(m + 1) * D] = t[...].T

    def mk_spec(m, V):
        last = (V - 1) // BL

        def imap(rb, m=m, last=last):
            return (0, jnp.minimum(m * n_rb + rb, last))

        return pl.BlockSpec((D, BL), imap)

    return pl.pallas_call(
        body,
        grid=(n_rb,),
        in_specs=[mk_spec(m, V) for m in range(4)],
        out_specs=pl.BlockSpec((BL, 4 * D), lambda rb: (rb, 0)),
        out_shape=jax.ShapeDtypeStruct((VP, 4 * D), jnp.float32),
    )


def kernel(inputs, table):
    Bb, H = inputs.shape
    V, D = table.shape
    tt = table.T
    table4 = _tc_relayout(V, D)(tt, tt, tt, tt)
    out2d = _build(Bb, H, V, D)(inputs.T, table4)
    return out2d.reshape(H, D, Bb).transpose(2, 0, 1)
